# baseline (speedup 1.0000x reference)
import functools

import jax
import jax.numpy as jnp
from jax import lax
from jax.experimental import pallas as pl
from jax.experimental.pallas import tpu as pltpu
from jax.experimental.pallas import tpu_sc as plsc

_INFO = plsc.get_sparse_core_info()
_NC = _INFO.num_cores
_NS = _INFO.num_subcores
_NW = _NC * _NS
_K = 80


@functools.partial(jax.jit, static_argnums=(0, 1, 2))
def _sc_aggregate(n_pad, n_edges, d, x, src, dst, zeros):
    edges_per_worker = n_edges // _NW
    steps = edges_per_worker // _K
    rows_per_tile = n_pad // _NS

    mesh = plsc.VectorSubcoreMesh(core_axis_name="c", subcore_axis_name="s")

    @functools.partial(
        pl.kernel,
        out_type=jax.ShapeDtypeStruct((_NC * n_pad, d), jnp.float32),
        mesh=mesh,
        scratch_types=[
            pltpu.VMEM((edges_per_worker,), jnp.int32),
            pltpu.VMEM((edges_per_worker,), jnp.int32),
            pltpu.VMEM((_K,), jnp.int32),
            pltpu.VMEM((_K,), jnp.int32),
            pltpu.VMEM((_K, d), jnp.float32),
            pltpu.VMEM((_K, d), jnp.float32),
            pltpu.VMEM_SHARED((n_pad, d), jnp.float32),
            pltpu.SemaphoreType.DMA,
            pltpu.SemaphoreType.DMA,
            pltpu.SemaphoreType.DMA,
        ],
    )
    def agg_kernel(x_hbm, src_hbm, dst_hbm, zeros_hbm, part_hbm,
                   src_v, dst_v, dstc_a, dstc_b, rows_a, rows_b, acc_sh,
                   sem_i, sem_a, sem_b):
        c = lax.axis_index("c")
        s = lax.axis_index("s")
        wid = s * _NC + c
        e0 = wid * edges_per_worker

        cp_s = pltpu.async_copy(src_hbm.at[pl.ds(e0, edges_per_worker)],
                                src_v, sem_i)
        cp_d = pltpu.async_copy(dst_hbm.at[pl.ds(e0, edges_per_worker)],
                                dst_v, sem_i)
        pltpu.sync_copy(zeros_hbm,
                        acc_sh.at[pl.ds(s * rows_per_tile, rows_per_tile)])
        cp_s.wait()
        cp_d.wait()
        plsc.subcore_barrier()

        def gather(i, buf, sem):
            pltpu.async_copy(x_hbm.at[src_v.at[pl.ds(i * _K, _K)]], buf, sem)

        def wait_gather(i, buf, sem):
            pltpu.make_async_copy(x_hbm.at[src_v.at[pl.ds(i * _K, _K)]], buf,
                                  sem).wait()

        def stage_dst(i, dstc):
            for k in range(_K // 16):
                dstc[pl.ds(16 * k, 16)] = dst_v[pl.ds(i * _K + 16 * k, 16)]

        def scat(buf, dstc):
            pltpu.sync_copy(buf, acc_sh.at[dstc], add=True)

        gather(0, rows_a, sem_a)

        def body(j, carry):
            i = 2 * j
            gather(i + 1, rows_b, sem_b)
            stage_dst(i, dstc_a)
            wait_gather(i, rows_a, sem_a)
            scat(rows_a, dstc_a)
            gather(i + 2, rows_a, sem_a)
            stage_dst(i + 1, dstc_b)
            wait_gather(i + 1, rows_b, sem_b)
            scat(rows_b, dstc_b)
            return carry

        lax.fori_loop(0, steps // 2, body, 0)
        stage_dst(steps - 1, dstc_a)
        wait_gather(steps - 1, rows_a, sem_a)
        scat(rows_a, dstc_a)
        plsc.subcore_barrier()

        off = c * n_pad + s * rows_per_tile
        pltpu.sync_copy(acc_sh.at[pl.ds(s * rows_per_tile, rows_per_tile)],
                        part_hbm.at[pl.ds(off, rows_per_tile)])

    return agg_kernel(x, src, dst, zeros)


def _tc_selfloop_body(x_ref, slw_ref, o_ref):
    o_ref[...] = lax.dot_general(
        x_ref[...], slw_ref[...], (((1,), (1,)), ((), ())),
        preferred_element_type=jnp.float32)


def _tc_combine_body(p0_ref, p1_ref, sl_ref, w_ref, o_ref):
    agg = p0_ref[0] + p1_ref[0]
    o_ref[...] = (
        jnp.dot(agg, w_ref[...], preferred_element_type=jnp.float32)
        + sl_ref[...]
    )


def kernel(x, edge_index, edge_type, weight, self_loop_w):
    n_nodes, d = x.shape
    n_edges = edge_index.shape[1]
    n_pad = ((n_nodes + 8 * _NS - 1) // (8 * _NS)) * (8 * _NS)
    zeros = jnp.zeros((n_pad // _NS, d), jnp.float32)

    blk = 2000
    grid = n_nodes // blk

    selfloop = pl.pallas_call(
        _tc_selfloop_body,
        grid=(grid,),
        in_specs=[
            pl.BlockSpec((blk, d), lambda i: (i, 0)),
            pl.BlockSpec((d, d), lambda i: (0, 0)),
        ],
        out_specs=pl.BlockSpec((blk, d), lambda i: (i, 0)),
        out_shape=jax.ShapeDtypeStruct((n_nodes, d), jnp.float32),
    )(x, self_loop_w)

    part = _sc_aggregate(n_pad, n_edges, d, x, edge_index[0], edge_index[1],
                         zeros)
    part3 = part.reshape(_NC, n_pad, d)

    out = pl.pallas_call(
        _tc_combine_body,
        grid=(grid,),
        in_specs=[
            pl.BlockSpec((1, blk, d), lambda i: (0, i, 0)),
            pl.BlockSpec((1, blk, d), lambda i: (1, i, 0)),
            pl.BlockSpec((blk, d), lambda i: (i, 0)),
            pl.BlockSpec((d, d), lambda i: (0, 0)),
        ],
        out_specs=pl.BlockSpec((blk, d), lambda i: (i, 0)),
        out_shape=jax.ShapeDtypeStruct((n_nodes, d), jnp.float32),
    )(part3, part3, selfloop, weight[0])
    return out
